# Initial kernel scaffold; baseline (speedup 1.0000x reference)
import jax
import jax.numpy as jnp
from jax.experimental import pallas as pl
from jax.experimental.pallas import tpu as pltpu

B = 2048
D = 512
K = 16
RB = 256
NBLK = B // RB
FULL = 2 * B
R3 = 512
NB3 = B // R3
EPS = 1e-12


def _dist(g, na_col, nb_row):
    return jnp.sqrt(jnp.maximum(na_col + nb_row - 2.0 * g, 0.0) + EPS)


def _matmul_nt(a, b):
    return jax.lax.dot_general(
        a, b, (((1,), (1,)), ((), ())),
        preferred_element_type=jnp.float32,
        precision=jax.lax.Precision.HIGHEST)


def _rank_value(bits, kk):
    r = bits.shape[0]
    lo = jnp.zeros((r, 1), jnp.int32)
    hi = jnp.full((r, 1), jnp.int32(0x7F800000))

    def body(_, carry):
        lo, hi = carry
        mid = lo + ((hi - lo) >> 1)
        cnt = jnp.sum((bits <= mid).astype(jnp.float32), axis=1, keepdims=True)
        pred = cnt >= (kk + 1.0)
        return jnp.where(pred, lo, mid + 1), jnp.where(pred, mid, hi)

    lo, hi = jax.lax.fori_loop(0, 31, body, (lo, hi))
    return jax.lax.bitcast_convert_type(lo, jnp.float32)


def _lid(d, tkv, kk, self_mask):
    mask = d <= tkv
    terms = jnp.where(mask, jnp.log(jnp.where(mask, d / tkv + EPS, 1.0)), 0.0)
    total = jnp.sum(terms, axis=1)
    dself = jnp.sum(jnp.where(self_mask, d, 0.0), axis=1)
    self_term = jnp.log(dself / tkv[:, 0] + EPS)
    return -kk / (total - self_term)


def _extract(dmat, n_extract):
    r, n = dmat.shape
    iota = jax.lax.broadcasted_iota(jnp.int32, (r, n), 1)
    vals, idxs = [], []
    cur = dmat
    for _ in range(n_extract):
        mv = jnp.min(cur, axis=1, keepdims=True)
        cand = jnp.where(cur == mv, iota, n)
        mi = jnp.min(cand, axis=1, keepdims=True)
        vals.append(mv)
        idxs.append(mi)
        cur = jnp.where(iota == mi, jnp.inf, cur)
    return vals, idxs


def _img_kernel(ls_ref, x_ref, t_ref, xb_ref,
                logits_ref, lse_row_ref, diag_ref, s16_ref, s17_ref,
                idx16_ref, v32_ref, v512_ref, lse_col_ref,
                colm_ref, cols_ref):
    i = pl.program_id(0)
    X = x_ref[:]
    T = t_ref[:]
    xb = xb_ref[:]
    ls = ls_ref[0]
    nx = jnp.sum(X * X, axis=1)[None, :]
    nt = jnp.sum(T * T, axis=1)[None, :]
    nxb = jnp.sum(xb * xb, axis=1)[:, None]

    g_ii = _matmul_nt(xb, X)
    g_it = _matmul_nt(xb, T)
    d_ii = _dist(g_ii, nxb, nx)
    d_it = _dist(g_it, nxb, nt)

    logits = ls * g_it
    logits_ref[:] = logits

    m = jnp.max(logits, axis=1, keepdims=True)
    lse_row_ref[:] = m[:, 0] + jnp.log(jnp.sum(jnp.exp(logits - m), axis=1))

    @pl.when(i == 0)
    def _():
        colm_ref[:] = jnp.full((1, B), -jnp.inf, jnp.float32)
        cols_ref[:] = jnp.zeros((1, B), jnp.float32)

    bm = jnp.max(logits, axis=0, keepdims=True)
    newm = jnp.maximum(colm_ref[:], bm)
    news = (cols_ref[:] * jnp.exp(colm_ref[:] - newm)
            + jnp.sum(jnp.exp(logits - newm), axis=0, keepdims=True))
    colm_ref[:] = newm
    cols_ref[:] = news

    @pl.when(i == NBLK - 1)
    def _():
        lse_col_ref[:] = (newm + jnp.log(news))[0]

    rows = jax.lax.broadcasted_iota(jnp.int32, (RB, B), 0) + i * RB
    colsb = jax.lax.broadcasted_iota(jnp.int32, (RB, B), 1)
    self_mask = colsb == rows
    diag_ref[:] = jnp.sum(jnp.where(self_mask, logits, 0.0), axis=1)

    dfull = jnp.concatenate([d_ii, d_it], axis=1)
    vals, idxs = _extract(dfull, 18)
    s16_ref[:] = vals[16][:, 0]
    s17_ref[:] = vals[17][:, 0]
    grow = rows[:, :1]
    idxmat = jnp.concatenate(idxs[1:17], axis=1)
    idx16_ref[:] = idxmat - (idxmat > grow).astype(jnp.int32)

    bits = jax.lax.bitcast_convert_type(d_ii, jnp.int32)
    t32v = _rank_value(bits, 32)
    t512v = _rank_value(bits, 512)
    v32_ref[:] = _lid(d_ii, t32v, 32.0, self_mask)
    v512_ref[:] = _lid(d_ii, t512v, 512.0, self_mask)


def _txt_kernel(x_ref, t_ref, tb_ref, s16_ref, t32_ref, t512_ref):
    i = pl.program_id(0)
    X = x_ref[:]
    T = t_ref[:]
    tb = tb_ref[:]
    nx = jnp.sum(X * X, axis=1)[None, :]
    nt = jnp.sum(T * T, axis=1)[None, :]
    ntb = jnp.sum(tb * tb, axis=1)[:, None]

    g_ti = _matmul_nt(tb, X)
    g_tt = _matmul_nt(tb, T)
    d_ti = _dist(g_ti, ntb, nx)
    d_tt = _dist(g_tt, ntb, nt)

    dfull = jnp.concatenate([d_ti, d_tt], axis=1)
    vals, _ = _extract(dfull, 17)
    s16_ref[:] = vals[16][:, 0]

    rows = jax.lax.broadcasted_iota(jnp.int32, (RB, B), 0) + i * RB
    colsb = jax.lax.broadcasted_iota(jnp.int32, (RB, B), 1)
    self_mask = colsb == rows

    bits = jax.lax.bitcast_convert_type(d_tt, jnp.int32)
    t32v = _rank_value(bits, 32)
    t512v = _rank_value(bits, 512)
    t32_ref[:] = _lid(d_tt, t32v, 32.0, self_mask)
    t512_ref[:] = _lid(d_tt, t512v, 512.0, self_mask)


def _final_kernel(tf_ref, idx_ref, s17_ref, lser_ref, lsec_ref, diag_ref,
                  scores_ref, loss_ref, acc_ref):
    i = pl.program_id(0)

    @pl.when(i == 0)
    def _():
        acc_ref[0] = 0.0
        acc_ref[1] = 0.0
        acc_ref[2] = 0.0

    w = 1.0 / tf_ref[:]
    idx = idx_ref[:]
    iota = jax.lax.broadcasted_iota(jnp.int32, (R3, FULL), 1)
    asum = jnp.zeros((R3,), jnp.float32)
    for j in range(K):
        sel = iota == idx[:, j][:, None]
        asum = asum + jnp.sum(jnp.where(sel, w[None, :], 0.0), axis=1)
    scores = s17_ref[:] * asum * (1.0 / K)
    scores_ref[:] = scores

    acc_ref[0] += jnp.sum(lser_ref[:] - diag_ref[:])
    acc_ref[1] += jnp.sum(lsec_ref[:] - diag_ref[:])
    acc_ref[2] += jnp.sum(scores)

    @pl.when(i == NB3 - 1)
    def _():
        sim = (acc_ref[0] + acc_ref[1]) / (2.0 * B)
        loss_ref[0, 0] = sim + acc_ref[2] / B


def _row_spec(blk):
    return pl.BlockSpec((blk,), lambda i: (i,))


def kernel(image_features, text_features, logit_scale):
    X = image_features
    T = text_features
    ls = logit_scale.astype(jnp.float32)

    full_spec = pl.BlockSpec((B, D), lambda i: (0, 0))
    blk_spec = pl.BlockSpec((RB, D), lambda i: (i, 0))

    out1 = pl.pallas_call(
        _img_kernel,
        grid=(NBLK,),
        in_specs=[
            pl.BlockSpec(memory_space=pltpu.SMEM),
            full_spec, full_spec, blk_spec,
        ],
        out_specs=[
            pl.BlockSpec((RB, B), lambda i: (i, 0)),
            _row_spec(RB), _row_spec(RB),
            _row_spec(RB), _row_spec(RB),
            pl.BlockSpec((RB, K), lambda i: (i, 0)),
            _row_spec(RB), _row_spec(RB),
            pl.BlockSpec((B,), lambda i: (0,)),
        ],
        out_shape=[
            jax.ShapeDtypeStruct((B, B), jnp.float32),
            jax.ShapeDtypeStruct((B,), jnp.float32),
            jax.ShapeDtypeStruct((B,), jnp.float32),
            jax.ShapeDtypeStruct((B,), jnp.float32),
            jax.ShapeDtypeStruct((B,), jnp.float32),
            jax.ShapeDtypeStruct((B, K), jnp.int32),
            jax.ShapeDtypeStruct((B,), jnp.float32),
            jax.ShapeDtypeStruct((B,), jnp.float32),
            jax.ShapeDtypeStruct((B,), jnp.float32),
        ],
        scratch_shapes=[
            pltpu.VMEM((1, B), jnp.float32),
            pltpu.VMEM((1, B), jnp.float32),
        ],
        compiler_params=pltpu.CompilerParams(
            dimension_semantics=("arbitrary",)),
    )(ls, X, T, X)
    (logits, lse_row, diag, s16_img, s17, idx16, v32, v512, lse_col) = out1

    out2 = pl.pallas_call(
        _txt_kernel,
        grid=(NBLK,),
        in_specs=[full_spec, full_spec, blk_spec],
        out_specs=[_row_spec(RB), _row_spec(RB), _row_spec(RB)],
        out_shape=[
            jax.ShapeDtypeStruct((B,), jnp.float32),
            jax.ShapeDtypeStruct((B,), jnp.float32),
            jax.ShapeDtypeStruct((B,), jnp.float32),
        ],
        compiler_params=pltpu.CompilerParams(
            dimension_semantics=("arbitrary",)),
    )(X, T, T)
    s16_txt, t32, t512 = out2

    t_full = jnp.concatenate([s16_img, s16_txt])

    scores, loss2d = pl.pallas_call(
        _final_kernel,
        grid=(NB3,),
        in_specs=[
            pl.BlockSpec((FULL,), lambda i: (0,)),
            pl.BlockSpec((R3, K), lambda i: (i, 0)),
            _row_spec(R3), _row_spec(R3), _row_spec(R3), _row_spec(R3),
        ],
        out_specs=[
            _row_spec(R3),
            pl.BlockSpec((1, 1), lambda i: (0, 0)),
        ],
        out_shape=[
            jax.ShapeDtypeStruct((B,), jnp.float32),
            jax.ShapeDtypeStruct((1, 1), jnp.float32),
        ],
        scratch_shapes=[pltpu.SMEM((4,), jnp.float32)],
        compiler_params=pltpu.CompilerParams(
            dimension_semantics=("arbitrary",)),
    )(t_full, idx16, s17, lse_row, lse_col, diag)

    labels = jnp.arange(B, dtype=jnp.int32)
    return (loss2d.reshape(()), logits, labels, scores, v32, v512, t32, t512)


# speedup vs baseline: 11.3381x; 11.3381x over previous
import jax
import jax.numpy as jnp
from jax.experimental import pallas as pl
from jax.experimental.pallas import tpu as pltpu

B = 2048
D = 512
K = 16
RB = 256
NBLK = B // RB
FULL = 2 * B
R3 = 512
NB3 = B // R3
EPS = 1e-12


def _dist(g, na_col, nb_row):
    return jnp.sqrt(jnp.maximum(na_col + nb_row - 2.0 * g, 0.0) + EPS)


def _matmul_nt(a, b):
    return jax.lax.dot_general(
        a, b, (((1,), (1,)), ((), ())),
        preferred_element_type=jnp.float32,
        precision=jax.lax.Precision.HIGHEST)


def _rank_value(bits, kk):
    r = bits.shape[0]
    lo = jnp.zeros((r, 1), jnp.int32)
    hi = jnp.full((r, 1), jnp.int32(0x7F800000))

    def body(_, carry):
        lo, hi = carry
        mid = lo + ((hi - lo) >> 1)
        cnt = jnp.sum((bits <= mid).astype(jnp.float32), axis=1, keepdims=True)
        pred = cnt >= (kk + 1.0)
        return jnp.where(pred, lo, mid + 1), jnp.where(pred, mid, hi)

    lo, hi = jax.lax.fori_loop(0, 31, body, (lo, hi))
    return jax.lax.bitcast_convert_type(lo, jnp.float32)


def _lid(d, tkv, kk, self_mask):
    mask = d <= tkv
    terms = jnp.where(mask, jnp.log(jnp.where(mask, d / tkv + EPS, 1.0)), 0.0)
    total = jnp.sum(terms, axis=1)
    dself = jnp.sum(jnp.where(self_mask, d, 0.0), axis=1)
    self_term = jnp.log(dself / tkv[:, 0] + EPS)
    return -kk / (total - self_term)


def _extract(dmat, n_extract):
    r, n = dmat.shape
    iota = jax.lax.broadcasted_iota(jnp.int32, (r, n), 1)
    vals, idxs = [], []
    cur = dmat
    for _ in range(n_extract):
        mv = jnp.min(cur, axis=1, keepdims=True)
        cand = jnp.where(cur == mv, iota, n)
        mi = jnp.min(cand, axis=1, keepdims=True)
        vals.append(mv)
        idxs.append(mi)
        cur = jnp.where(iota == mi, jnp.inf, cur)
    return vals, idxs


def _img_kernel(ls_ref, x_ref, t_ref, xb_ref,
                logits_ref, lse_row_ref, diag_ref, s16_ref, s17_ref,
                idx16_ref, v32_ref, v512_ref, lse_col_ref,
                colm_ref, cols_ref):
    i = pl.program_id(0)
    X = x_ref[:]
    T = t_ref[:]
    xb = xb_ref[:]
    ls = ls_ref[0]
    nx = jnp.sum(X * X, axis=1)[None, :]
    nt = jnp.sum(T * T, axis=1)[None, :]
    nxb = jnp.sum(xb * xb, axis=1)[:, None]

    g_ii = _matmul_nt(xb, X)
    g_it = _matmul_nt(xb, T)
    d_ii = _dist(g_ii, nxb, nx)
    d_it = _dist(g_it, nxb, nt)

    logits = ls * g_it
    logits_ref[:] = logits

    m = jnp.max(logits, axis=1, keepdims=True)
    lse_row_ref[:] = m[:, 0] + jnp.log(jnp.sum(jnp.exp(logits - m), axis=1))

    @pl.when(i == 0)
    def _():
        colm_ref[:] = jnp.full((1, B), -jnp.inf, jnp.float32)
        cols_ref[:] = jnp.zeros((1, B), jnp.float32)

    bm = jnp.max(logits, axis=0, keepdims=True)
    newm = jnp.maximum(colm_ref[:], bm)
    news = (cols_ref[:] * jnp.exp(colm_ref[:] - newm)
            + jnp.sum(jnp.exp(logits - newm), axis=0, keepdims=True))
    colm_ref[:] = newm
    cols_ref[:] = news

    @pl.when(i == NBLK - 1)
    def _():
        lse_col_ref[:] = (newm + jnp.log(news))[0]

    rows = jax.lax.broadcasted_iota(jnp.int32, (RB, B), 0) + i * RB
    colsb = jax.lax.broadcasted_iota(jnp.int32, (RB, B), 1)
    self_mask = colsb == rows
    diag_ref[:] = jnp.sum(jnp.where(self_mask, logits, 0.0), axis=1)

    dfull = jnp.concatenate([d_ii, d_it], axis=1)
    vals, idxs = _extract(dfull, 18)
    s16_ref[:] = vals[16][:, 0]
    s17_ref[:] = vals[17][:, 0]
    grow = rows[:, :1]
    idxmat = jnp.concatenate(idxs[1:17], axis=1)
    idx16_ref[:] = idxmat - (idxmat > grow).astype(jnp.int32)

    bits = jax.lax.bitcast_convert_type(d_ii, jnp.int32)
    t32v = _rank_value(bits, 32)
    t512v = _rank_value(bits, 512)
    v32_ref[:] = _lid(d_ii, t32v, 32.0, self_mask)
    v512_ref[:] = _lid(d_ii, t512v, 512.0, self_mask)


def _txt_kernel(x_ref, t_ref, tb_ref, s16_ref, t32_ref, t512_ref):
    i = pl.program_id(0)
    X = x_ref[:]
    T = t_ref[:]
    tb = tb_ref[:]
    nx = jnp.sum(X * X, axis=1)[None, :]
    nt = jnp.sum(T * T, axis=1)[None, :]
    ntb = jnp.sum(tb * tb, axis=1)[:, None]

    g_ti = _matmul_nt(tb, X)
    g_tt = _matmul_nt(tb, T)
    d_ti = _dist(g_ti, ntb, nx)
    d_tt = _dist(g_tt, ntb, nt)

    dfull = jnp.concatenate([d_ti, d_tt], axis=1)
    vals, _ = _extract(dfull, 17)
    s16_ref[:] = vals[16][:, 0]

    rows = jax.lax.broadcasted_iota(jnp.int32, (RB, B), 0) + i * RB
    colsb = jax.lax.broadcasted_iota(jnp.int32, (RB, B), 1)
    self_mask = colsb == rows

    bits = jax.lax.bitcast_convert_type(d_tt, jnp.int32)
    t32v = _rank_value(bits, 32)
    t512v = _rank_value(bits, 512)
    t32_ref[:] = _lid(d_tt, t32v, 32.0, self_mask)
    t512_ref[:] = _lid(d_tt, t512v, 512.0, self_mask)


def _final_kernel(tf_ref, idx_ref, s17_ref, lser_ref, lsec_ref, diag_ref,
                  scores_ref, loss_ref, acc_ref):
    i = pl.program_id(0)

    @pl.when(i == 0)
    def _():
        acc_ref[0] = 0.0
        acc_ref[1] = 0.0
        acc_ref[2] = 0.0

    w = 1.0 / tf_ref[:]
    idx = idx_ref[:]
    iota = jax.lax.broadcasted_iota(jnp.int32, (R3, FULL), 1)
    asum = jnp.zeros((R3,), jnp.float32)
    for j in range(K):
        sel = iota == idx[:, j][:, None]
        asum = asum + jnp.sum(jnp.where(sel, w[None, :], 0.0), axis=1)
    scores = s17_ref[:] * asum * (1.0 / K)
    scores_ref[:] = scores

    acc_ref[0] += jnp.sum(lser_ref[:] - diag_ref[:])
    acc_ref[1] += jnp.sum(lsec_ref[:] - diag_ref[:])
    acc_ref[2] += jnp.sum(scores)

    @pl.when(i == NB3 - 1)
    def _():
        sim = (acc_ref[0] + acc_ref[1]) / (2.0 * B)
        loss_ref[:, :] = jnp.full((1, 1), sim + acc_ref[2] / B, jnp.float32)


def _row_spec(blk):
    return pl.BlockSpec((blk,), lambda i: (i,))


def kernel(image_features, text_features, logit_scale):
    X = image_features
    T = text_features
    ls = logit_scale.astype(jnp.float32)

    full_spec = pl.BlockSpec((B, D), lambda i: (0, 0))
    blk_spec = pl.BlockSpec((RB, D), lambda i: (i, 0))

    out1 = pl.pallas_call(
        _img_kernel,
        grid=(NBLK,),
        in_specs=[
            pl.BlockSpec(memory_space=pltpu.SMEM),
            full_spec, full_spec, blk_spec,
        ],
        out_specs=[
            pl.BlockSpec((RB, B), lambda i: (i, 0)),
            _row_spec(RB), _row_spec(RB),
            _row_spec(RB), _row_spec(RB),
            pl.BlockSpec((RB, K), lambda i: (i, 0)),
            _row_spec(RB), _row_spec(RB),
            pl.BlockSpec((B,), lambda i: (0,)),
        ],
        out_shape=[
            jax.ShapeDtypeStruct((B, B), jnp.float32),
            jax.ShapeDtypeStruct((B,), jnp.float32),
            jax.ShapeDtypeStruct((B,), jnp.float32),
            jax.ShapeDtypeStruct((B,), jnp.float32),
            jax.ShapeDtypeStruct((B,), jnp.float32),
            jax.ShapeDtypeStruct((B, K), jnp.int32),
            jax.ShapeDtypeStruct((B,), jnp.float32),
            jax.ShapeDtypeStruct((B,), jnp.float32),
            jax.ShapeDtypeStruct((B,), jnp.float32),
        ],
        scratch_shapes=[
            pltpu.VMEM((1, B), jnp.float32),
            pltpu.VMEM((1, B), jnp.float32),
        ],
        compiler_params=pltpu.CompilerParams(
            dimension_semantics=("arbitrary",)),
    )(ls, X, T, X)
    (logits, lse_row, diag, s16_img, s17, idx16, v32, v512, lse_col) = out1

    out2 = pl.pallas_call(
        _txt_kernel,
        grid=(NBLK,),
        in_specs=[full_spec, full_spec, blk_spec],
        out_specs=[_row_spec(RB), _row_spec(RB), _row_spec(RB)],
        out_shape=[
            jax.ShapeDtypeStruct((B,), jnp.float32),
            jax.ShapeDtypeStruct((B,), jnp.float32),
            jax.ShapeDtypeStruct((B,), jnp.float32),
        ],
        compiler_params=pltpu.CompilerParams(
            dimension_semantics=("arbitrary",)),
    )(X, T, T)
    s16_txt, t32, t512 = out2

    t_full = jnp.concatenate([s16_img, s16_txt])

    scores, loss2d = pl.pallas_call(
        _final_kernel,
        grid=(NB3,),
        in_specs=[
            pl.BlockSpec((FULL,), lambda i: (0,)),
            pl.BlockSpec((R3, K), lambda i: (i, 0)),
            _row_spec(R3), _row_spec(R3), _row_spec(R3), _row_spec(R3),
        ],
        out_specs=[
            _row_spec(R3),
            pl.BlockSpec((1, 1), lambda i: (0, 0)),
        ],
        out_shape=[
            jax.ShapeDtypeStruct((B,), jnp.float32),
            jax.ShapeDtypeStruct((1, 1), jnp.float32),
        ],
        scratch_shapes=[pltpu.SMEM((4,), jnp.float32)],
        compiler_params=pltpu.CompilerParams(
            dimension_semantics=("arbitrary",)),
    )(t_full, idx16, s17, lse_row, lse_col, diag)

    labels = jnp.arange(B, dtype=jnp.int32)
    return (loss2d.reshape(()), logits, labels, scores, v32, v512, t32, t512)


# speedup vs baseline: 14.3576x; 1.2663x over previous
import jax
import jax.numpy as jnp
from jax.experimental import pallas as pl
from jax.experimental.pallas import tpu as pltpu

B = 2048
D = 512
K = 16
RB = 256
NBLK = B // RB
FULL = 2 * B
R3 = 512
NB3 = B // R3
EPS = 1e-12


def _dist(g, na_col, nb_row):
    return jnp.sqrt(jnp.maximum(na_col + nb_row - 2.0 * g, 0.0) + EPS)


def _matmul_nt(a, b):
    return jax.lax.dot_general(
        a, b, (((1,), (1,)), ((), ())),
        preferred_element_type=jnp.float32,
        precision=jax.lax.Precision.HIGHEST)


_LO_BITS = 0x358637BD
_HI_BITS = 0x40100000
_N_BISECT = 28


def _rank_value2(bits, k_a, k_b):
    r = bits.shape[0]
    lo_a = jnp.full((r, 1), jnp.int32(_LO_BITS))
    hi_a = jnp.full((r, 1), jnp.int32(_HI_BITS))
    lo_b = lo_a
    hi_b = hi_a

    def body(_, carry):
        lo_a, hi_a, lo_b, hi_b = carry
        mid_a = lo_a + ((hi_a - lo_a) >> 1)
        mid_b = lo_b + ((hi_b - lo_b) >> 1)
        cnt_a = jnp.sum((bits <= mid_a).astype(jnp.float32), axis=1,
                        keepdims=True)
        cnt_b = jnp.sum((bits <= mid_b).astype(jnp.float32), axis=1,
                        keepdims=True)
        p_a = cnt_a >= (k_a + 1.0)
        p_b = cnt_b >= (k_b + 1.0)
        return (jnp.where(p_a, lo_a, mid_a + 1), jnp.where(p_a, mid_a, hi_a),
                jnp.where(p_b, lo_b, mid_b + 1), jnp.where(p_b, mid_b, hi_b))

    lo_a, _, lo_b, _ = jax.lax.fori_loop(0, _N_BISECT, body,
                                         (lo_a, hi_a, lo_b, hi_b))
    return (jax.lax.bitcast_convert_type(lo_a, jnp.float32),
            jax.lax.bitcast_convert_type(lo_b, jnp.float32))


def _lid2(d, self_mask):
    bits = jax.lax.bitcast_convert_type(d, jnp.int32)
    t32v, t512v = _rank_value2(bits, 32.0, 512.0)
    logd = jnp.log(d)
    s32 = jnp.sum(jnp.where(d <= t32v, logd, 0.0), axis=1)
    c32 = jnp.sum((d <= t32v).astype(jnp.float32), axis=1)
    s512 = jnp.sum(jnp.where(d <= t512v, logd, 0.0), axis=1)
    c512 = jnp.sum((d <= t512v).astype(jnp.float32), axis=1)
    dself = jnp.sum(jnp.where(self_mask, d, 0.0), axis=1)
    lself = jnp.log(dself)
    tot32 = (s32 - lself) - (c32 - 1.0) * jnp.log(t32v[:, 0])
    tot512 = (s512 - lself) - (c512 - 1.0) * jnp.log(t512v[:, 0])
    return -32.0 / tot32, -512.0 / tot512


def _extract(dmat, n_extract, want_idx):
    r, n = dmat.shape
    iota = jax.lax.broadcasted_iota(jnp.int32, (r, n), 1)
    vals, idxs = [], []
    cur = dmat
    for _ in range(n_extract):
        mv = jnp.min(cur, axis=1, keepdims=True)
        mask = cur == mv
        vals.append(mv)
        if want_idx:
            cand = jnp.where(mask, iota, n)
            idxs.append(jnp.min(cand, axis=1, keepdims=True))
        cur = jnp.where(mask, jnp.inf, cur)
    return vals, idxs


def _img_kernel(ls_ref, x_ref, t_ref, xb_ref,
                logits_ref, lse_row_ref, diag_ref, s16_ref, s17_ref,
                idx16_ref, v32_ref, v512_ref, lse_col_ref,
                colm_ref, cols_ref):
    i = pl.program_id(0)
    X = x_ref[:]
    T = t_ref[:]
    xb = xb_ref[:]
    ls = ls_ref[0]
    nx = jnp.sum(X * X, axis=1)[None, :]
    nt = jnp.sum(T * T, axis=1)[None, :]
    nxb = jnp.sum(xb * xb, axis=1)[:, None]

    g_ii = _matmul_nt(xb, X)
    g_it = _matmul_nt(xb, T)
    d_ii = _dist(g_ii, nxb, nx)
    d_it = _dist(g_it, nxb, nt)

    logits = ls * g_it
    logits_ref[:] = logits

    m = jnp.max(logits, axis=1, keepdims=True)
    lse_row_ref[:] = m[:, 0] + jnp.log(jnp.sum(jnp.exp(logits - m), axis=1))

    @pl.when(i == 0)
    def _():
        colm_ref[:] = jnp.full((1, B), -jnp.inf, jnp.float32)
        cols_ref[:] = jnp.zeros((1, B), jnp.float32)

    bm = jnp.max(logits, axis=0, keepdims=True)
    newm = jnp.maximum(colm_ref[:], bm)
    news = (cols_ref[:] * jnp.exp(colm_ref[:] - newm)
            + jnp.sum(jnp.exp(logits - newm), axis=0, keepdims=True))
    colm_ref[:] = newm
    cols_ref[:] = news

    @pl.when(i == NBLK - 1)
    def _():
        lse_col_ref[:] = (newm + jnp.log(news))[0]

    rows = jax.lax.broadcasted_iota(jnp.int32, (RB, B), 0) + i * RB
    colsb = jax.lax.broadcasted_iota(jnp.int32, (RB, B), 1)
    self_mask = colsb == rows
    diag_ref[:] = jnp.sum(jnp.where(self_mask, logits, 0.0), axis=1)

    dfull = jnp.concatenate([d_ii, d_it], axis=1)
    vals, idxs = _extract(dfull, 18, True)
    s16_ref[:] = vals[16][:, 0]
    s17_ref[:] = vals[17][:, 0]
    grow = rows[:, :1]
    idxmat = jnp.concatenate(idxs[1:17], axis=1)
    idx16_ref[:] = idxmat - (idxmat > grow).astype(jnp.int32)

    v32, v512 = _lid2(d_ii, self_mask)
    v32_ref[:] = v32
    v512_ref[:] = v512


def _txt_kernel(x_ref, t_ref, tb_ref, s16_ref, t32_ref, t512_ref):
    i = pl.program_id(0)
    X = x_ref[:]
    T = t_ref[:]
    tb = tb_ref[:]
    nx = jnp.sum(X * X, axis=1)[None, :]
    nt = jnp.sum(T * T, axis=1)[None, :]
    ntb = jnp.sum(tb * tb, axis=1)[:, None]

    g_ti = _matmul_nt(tb, X)
    g_tt = _matmul_nt(tb, T)
    d_ti = _dist(g_ti, ntb, nx)
    d_tt = _dist(g_tt, ntb, nt)

    dfull = jnp.concatenate([d_ti, d_tt], axis=1)
    vals, _ = _extract(dfull, 17, False)
    s16_ref[:] = vals[16][:, 0]

    rows = jax.lax.broadcasted_iota(jnp.int32, (RB, B), 0) + i * RB
    colsb = jax.lax.broadcasted_iota(jnp.int32, (RB, B), 1)
    self_mask = colsb == rows

    t32, t512 = _lid2(d_tt, self_mask)
    t32_ref[:] = t32
    t512_ref[:] = t512


def _final_kernel(tf_ref, idx_ref, s17_ref, lser_ref, lsec_ref, diag_ref,
                  scores_ref, loss_ref, acc_ref):
    i = pl.program_id(0)

    @pl.when(i == 0)
    def _():
        acc_ref[0] = 0.0
        acc_ref[1] = 0.0
        acc_ref[2] = 0.0

    w = 1.0 / tf_ref[:]
    idx = idx_ref[:]
    iota = jax.lax.broadcasted_iota(jnp.int32, (R3, FULL), 1)
    asum = jnp.zeros((R3,), jnp.float32)
    for j in range(K):
        sel = iota == idx[:, j][:, None]
        asum = asum + jnp.sum(jnp.where(sel, w[None, :], 0.0), axis=1)
    scores = s17_ref[:] * asum * (1.0 / K)
    scores_ref[:] = scores

    acc_ref[0] += jnp.sum(lser_ref[:] - diag_ref[:])
    acc_ref[1] += jnp.sum(lsec_ref[:] - diag_ref[:])
    acc_ref[2] += jnp.sum(scores)

    @pl.when(i == NB3 - 1)
    def _():
        sim = (acc_ref[0] + acc_ref[1]) / (2.0 * B)
        loss_ref[:, :] = jnp.full((1, 1), sim + acc_ref[2] / B, jnp.float32)


def _row_spec(blk):
    return pl.BlockSpec((blk,), lambda i: (i,))


def kernel(image_features, text_features, logit_scale):
    X = image_features
    T = text_features
    ls = logit_scale.astype(jnp.float32)

    full_spec = pl.BlockSpec((B, D), lambda i: (0, 0))
    blk_spec = pl.BlockSpec((RB, D), lambda i: (i, 0))

    out1 = pl.pallas_call(
        _img_kernel,
        grid=(NBLK,),
        in_specs=[
            pl.BlockSpec(memory_space=pltpu.SMEM),
            full_spec, full_spec, blk_spec,
        ],
        out_specs=[
            pl.BlockSpec((RB, B), lambda i: (i, 0)),
            _row_spec(RB), _row_spec(RB),
            _row_spec(RB), _row_spec(RB),
            pl.BlockSpec((RB, K), lambda i: (i, 0)),
            _row_spec(RB), _row_spec(RB),
            pl.BlockSpec((B,), lambda i: (0,)),
        ],
        out_shape=[
            jax.ShapeDtypeStruct((B, B), jnp.float32),
            jax.ShapeDtypeStruct((B,), jnp.float32),
            jax.ShapeDtypeStruct((B,), jnp.float32),
            jax.ShapeDtypeStruct((B,), jnp.float32),
            jax.ShapeDtypeStruct((B,), jnp.float32),
            jax.ShapeDtypeStruct((B, K), jnp.int32),
            jax.ShapeDtypeStruct((B,), jnp.float32),
            jax.ShapeDtypeStruct((B,), jnp.float32),
            jax.ShapeDtypeStruct((B,), jnp.float32),
        ],
        scratch_shapes=[
            pltpu.VMEM((1, B), jnp.float32),
            pltpu.VMEM((1, B), jnp.float32),
        ],
        compiler_params=pltpu.CompilerParams(
            dimension_semantics=("arbitrary",)),
    )(ls, X, T, X)
    (logits, lse_row, diag, s16_img, s17, idx16, v32, v512, lse_col) = out1

    out2 = pl.pallas_call(
        _txt_kernel,
        grid=(NBLK,),
        in_specs=[full_spec, full_spec, blk_spec],
        out_specs=[_row_spec(RB), _row_spec(RB), _row_spec(RB)],
        out_shape=[
            jax.ShapeDtypeStruct((B,), jnp.float32),
            jax.ShapeDtypeStruct((B,), jnp.float32),
            jax.ShapeDtypeStruct((B,), jnp.float32),
        ],
        compiler_params=pltpu.CompilerParams(
            dimension_semantics=("arbitrary",)),
    )(X, T, T)
    s16_txt, t32, t512 = out2

    t_full = jnp.concatenate([s16_img, s16_txt])

    scores, loss2d = pl.pallas_call(
        _final_kernel,
        grid=(NB3,),
        in_specs=[
            pl.BlockSpec((FULL,), lambda i: (0,)),
            pl.BlockSpec((R3, K), lambda i: (i, 0)),
            _row_spec(R3), _row_spec(R3), _row_spec(R3), _row_spec(R3),
        ],
        out_specs=[
            _row_spec(R3),
            pl.BlockSpec((1, 1), lambda i: (0, 0)),
        ],
        out_shape=[
            jax.ShapeDtypeStruct((B,), jnp.float32),
            jax.ShapeDtypeStruct((1, 1), jnp.float32),
        ],
        scratch_shapes=[pltpu.SMEM((4,), jnp.float32)],
        compiler_params=pltpu.CompilerParams(
            dimension_semantics=("arbitrary",)),
    )(t_full, idx16, s17, lse_row, lse_col, diag)

    labels = jnp.arange(B, dtype=jnp.int32)
    return (loss2d.reshape(()), logits, labels, scores, v32, v512, t32, t512)


# speedup vs baseline: 16.9178x; 1.1783x over previous
import jax
import jax.numpy as jnp
from jax.experimental import pallas as pl
from jax.experimental.pallas import tpu as pltpu

B = 2048
D = 512
K = 16
RB = 256
NBLK = B // RB
FULL = 2 * B
R3 = 512
NB3 = B // R3
EPS = 1e-12


def _dist(g, na_col, nb_row):
    return jnp.sqrt(jnp.maximum(na_col + nb_row - 2.0 * g, 0.0) + EPS)


def _matmul_nt(a, b):
    return jax.lax.dot_general(
        a, b, (((1,), (1,)), ((), ())),
        preferred_element_type=jnp.float32,
        precision=jax.lax.Precision.DEFAULT)


_LO_BITS = 0x358637BD
_HI_BITS = 0x40100000
_N_BISECT = 28


def _rank_value2(bits, k_a, k_b):
    r = bits.shape[0]
    lo_a = jnp.full((r, 1), jnp.int32(_LO_BITS))
    hi_a = jnp.full((r, 1), jnp.int32(_HI_BITS))
    lo_b = lo_a
    hi_b = hi_a

    def body(_, carry):
        lo_a, hi_a, lo_b, hi_b = carry
        mid_a = lo_a + ((hi_a - lo_a) >> 1)
        mid_b = lo_b + ((hi_b - lo_b) >> 1)
        cnt_a = jnp.sum((bits <= mid_a).astype(jnp.float32), axis=1,
                        keepdims=True)
        cnt_b = jnp.sum((bits <= mid_b).astype(jnp.float32), axis=1,
                        keepdims=True)
        p_a = cnt_a >= (k_a + 1.0)
        p_b = cnt_b >= (k_b + 1.0)
        return (jnp.where(p_a, lo_a, mid_a + 1), jnp.where(p_a, mid_a, hi_a),
                jnp.where(p_b, lo_b, mid_b + 1), jnp.where(p_b, mid_b, hi_b))

    lo_a, _, lo_b, _ = jax.lax.fori_loop(0, _N_BISECT, body,
                                         (lo_a, hi_a, lo_b, hi_b))
    return (jax.lax.bitcast_convert_type(lo_a, jnp.float32),
            jax.lax.bitcast_convert_type(lo_b, jnp.float32))


def _lid2(d, self_mask):
    bits = jax.lax.bitcast_convert_type(d, jnp.int32)
    t32v, t512v = _rank_value2(bits, 32.0, 512.0)
    logd = jnp.log(d)
    s32 = jnp.sum(jnp.where(d <= t32v, logd, 0.0), axis=1)
    c32 = jnp.sum((d <= t32v).astype(jnp.float32), axis=1)
    s512 = jnp.sum(jnp.where(d <= t512v, logd, 0.0), axis=1)
    c512 = jnp.sum((d <= t512v).astype(jnp.float32), axis=1)
    dself = jnp.sum(jnp.where(self_mask, d, 0.0), axis=1)
    lself = jnp.log(dself)
    tot32 = (s32 - lself) - (c32 - 1.0) * jnp.log(t32v[:, 0])
    tot512 = (s512 - lself) - (c512 - 1.0) * jnp.log(t512v[:, 0])
    return -32.0 / tot32, -512.0 / tot512


def _extract(dmat, n_extract, want_idx):
    r, n = dmat.shape
    iota = jax.lax.broadcasted_iota(jnp.int32, (r, n), 1)
    vals, idxs = [], []
    cur = dmat
    for _ in range(n_extract):
        mv = jnp.min(cur, axis=1, keepdims=True)
        mask = cur == mv
        vals.append(mv)
        if want_idx:
            cand = jnp.where(mask, iota, n)
            idxs.append(jnp.min(cand, axis=1, keepdims=True))
        cur = jnp.where(mask, jnp.inf, cur)
    return vals, idxs


def _img_kernel(ls_ref, x_ref, t_ref, xb_ref,
                logits_ref, lse_row_ref, diag_ref, s16_ref, s17_ref,
                idx16_ref, v32_ref, v512_ref, lse_col_ref,
                colm_ref, cols_ref):
    i = pl.program_id(0)
    X = x_ref[:]
    T = t_ref[:]
    xb = xb_ref[:]
    ls = ls_ref[0]
    nx = jnp.sum(X * X, axis=1)[None, :]
    nt = jnp.sum(T * T, axis=1)[None, :]
    nxb = jnp.sum(xb * xb, axis=1)[:, None]

    g_ii = _matmul_nt(xb, X)
    g_it = _matmul_nt(xb, T)
    d_ii = _dist(g_ii, nxb, nx)
    d_it = _dist(g_it, nxb, nt)

    logits = ls * g_it
    logits_ref[:] = logits

    m = jnp.max(logits, axis=1, keepdims=True)
    lse_row_ref[:] = m[:, 0] + jnp.log(jnp.sum(jnp.exp(logits - m), axis=1))

    @pl.when(i == 0)
    def _():
        colm_ref[:] = jnp.full((1, B), -jnp.inf, jnp.float32)
        cols_ref[:] = jnp.zeros((1, B), jnp.float32)

    bm = jnp.max(logits, axis=0, keepdims=True)
    newm = jnp.maximum(colm_ref[:], bm)
    news = (cols_ref[:] * jnp.exp(colm_ref[:] - newm)
            + jnp.sum(jnp.exp(logits - newm), axis=0, keepdims=True))
    colm_ref[:] = newm
    cols_ref[:] = news

    @pl.when(i == NBLK - 1)
    def _():
        lse_col_ref[:] = (newm + jnp.log(news))[0]

    rows = jax.lax.broadcasted_iota(jnp.int32, (RB, B), 0) + i * RB
    colsb = jax.lax.broadcasted_iota(jnp.int32, (RB, B), 1)
    self_mask = colsb == rows
    diag_ref[:] = jnp.sum(jnp.where(self_mask, logits, 0.0), axis=1)

    dfull = jnp.concatenate([jnp.where(self_mask, jnp.inf, d_ii), d_it],
                            axis=1)
    vals, idxs = _extract(dfull, 17, True)
    s16_ref[:] = vals[15][:, 0]
    s17_ref[:] = vals[16][:, 0]
    grow = rows[:, :1]
    idxmat = jnp.concatenate(idxs[0:16], axis=1)
    idx16_ref[:] = idxmat - (idxmat > grow).astype(jnp.int32)

    v32, v512 = _lid2(d_ii, self_mask)
    v32_ref[:] = v32
    v512_ref[:] = v512


def _txt_kernel(x_ref, t_ref, tb_ref, s16_ref, t32_ref, t512_ref):
    i = pl.program_id(0)
    X = x_ref[:]
    T = t_ref[:]
    tb = tb_ref[:]
    nx = jnp.sum(X * X, axis=1)[None, :]
    nt = jnp.sum(T * T, axis=1)[None, :]
    ntb = jnp.sum(tb * tb, axis=1)[:, None]

    g_ti = _matmul_nt(tb, X)
    g_tt = _matmul_nt(tb, T)
    d_ti = _dist(g_ti, ntb, nx)
    d_tt = _dist(g_tt, ntb, nt)

    rows = jax.lax.broadcasted_iota(jnp.int32, (RB, B), 0) + i * RB
    colsb = jax.lax.broadcasted_iota(jnp.int32, (RB, B), 1)
    self_mask = colsb == rows

    dfull = jnp.concatenate([d_ti, jnp.where(self_mask, jnp.inf, d_tt)],
                            axis=1)
    vals, _ = _extract(dfull, 16, False)
    s16_ref[:] = vals[15][:, 0]

    t32, t512 = _lid2(d_tt, self_mask)
    t32_ref[:] = t32
    t512_ref[:] = t512


def _final_kernel(tf_ref, idx_ref, s17_ref, lser_ref, lsec_ref, diag_ref,
                  scores_ref, loss_ref, acc_ref):
    i = pl.program_id(0)

    @pl.when(i == 0)
    def _():
        acc_ref[0] = 0.0
        acc_ref[1] = 0.0
        acc_ref[2] = 0.0

    w = 1.0 / tf_ref[:]
    idx = idx_ref[:]
    iota = jax.lax.broadcasted_iota(jnp.int32, (R3, FULL), 1)
    asum = jnp.zeros((R3,), jnp.float32)
    for j in range(K):
        sel = iota == idx[:, j][:, None]
        asum = asum + jnp.sum(jnp.where(sel, w[None, :], 0.0), axis=1)
    scores = s17_ref[:] * asum * (1.0 / K)
    scores_ref[:] = scores

    acc_ref[0] += jnp.sum(lser_ref[:] - diag_ref[:])
    acc_ref[1] += jnp.sum(lsec_ref[:] - diag_ref[:])
    acc_ref[2] += jnp.sum(scores)

    @pl.when(i == NB3 - 1)
    def _():
        sim = (acc_ref[0] + acc_ref[1]) / (2.0 * B)
        loss_ref[:, :] = jnp.full((1, 1), sim + acc_ref[2] / B, jnp.float32)


def _row_spec(blk):
    return pl.BlockSpec((blk,), lambda i: (i,))


def kernel(image_features, text_features, logit_scale):
    X = image_features
    T = text_features
    ls = logit_scale.astype(jnp.float32)

    full_spec = pl.BlockSpec((B, D), lambda i: (0, 0))
    blk_spec = pl.BlockSpec((RB, D), lambda i: (i, 0))

    out1 = pl.pallas_call(
        _img_kernel,
        grid=(NBLK,),
        in_specs=[
            pl.BlockSpec(memory_space=pltpu.SMEM),
            full_spec, full_spec, blk_spec,
        ],
        out_specs=[
            pl.BlockSpec((RB, B), lambda i: (i, 0)),
            _row_spec(RB), _row_spec(RB),
            _row_spec(RB), _row_spec(RB),
            pl.BlockSpec((RB, K), lambda i: (i, 0)),
            _row_spec(RB), _row_spec(RB),
            pl.BlockSpec((B,), lambda i: (0,)),
        ],
        out_shape=[
            jax.ShapeDtypeStruct((B, B), jnp.float32),
            jax.ShapeDtypeStruct((B,), jnp.float32),
            jax.ShapeDtypeStruct((B,), jnp.float32),
            jax.ShapeDtypeStruct((B,), jnp.float32),
            jax.ShapeDtypeStruct((B,), jnp.float32),
            jax.ShapeDtypeStruct((B, K), jnp.int32),
            jax.ShapeDtypeStruct((B,), jnp.float32),
            jax.ShapeDtypeStruct((B,), jnp.float32),
            jax.ShapeDtypeStruct((B,), jnp.float32),
        ],
        scratch_shapes=[
            pltpu.VMEM((1, B), jnp.float32),
            pltpu.VMEM((1, B), jnp.float32),
        ],
        compiler_params=pltpu.CompilerParams(
            dimension_semantics=("arbitrary",)),
    )(ls, X, T, X)
    (logits, lse_row, diag, s16_img, s17, idx16, v32, v512, lse_col) = out1

    out2 = pl.pallas_call(
        _txt_kernel,
        grid=(NBLK,),
        in_specs=[full_spec, full_spec, blk_spec],
        out_specs=[_row_spec(RB), _row_spec(RB), _row_spec(RB)],
        out_shape=[
            jax.ShapeDtypeStruct((B,), jnp.float32),
            jax.ShapeDtypeStruct((B,), jnp.float32),
            jax.ShapeDtypeStruct((B,), jnp.float32),
        ],
        compiler_params=pltpu.CompilerParams(
            dimension_semantics=("arbitrary",)),
    )(X, T, T)
    s16_txt, t32, t512 = out2

    t_full = jnp.concatenate([s16_img, s16_txt])

    scores, loss2d = pl.pallas_call(
        _final_kernel,
        grid=(NB3,),
        in_specs=[
            pl.BlockSpec((FULL,), lambda i: (0,)),
            pl.BlockSpec((R3, K), lambda i: (i, 0)),
            _row_spec(R3), _row_spec(R3), _row_spec(R3), _row_spec(R3),
        ],
        out_specs=[
            _row_spec(R3),
            pl.BlockSpec((1, 1), lambda i: (0, 0)),
        ],
        out_shape=[
            jax.ShapeDtypeStruct((B,), jnp.float32),
            jax.ShapeDtypeStruct((1, 1), jnp.float32),
        ],
        scratch_shapes=[pltpu.SMEM((4,), jnp.float32)],
        compiler_params=pltpu.CompilerParams(
            dimension_semantics=("arbitrary",)),
    )(t_full, idx16, s17, lse_row, lse_col, diag)

    labels = jnp.arange(B, dtype=jnp.int32)
    return (loss2d.reshape(()), logits, labels, scores, v32, v512, t32, t512)


# speedup vs baseline: 20.8316x; 1.2313x over previous
import jax
import jax.numpy as jnp
from jax.experimental import pallas as pl
from jax.experimental.pallas import tpu as pltpu

B = 2048
D = 512
K = 16
RB = 256
NBLK = B // RB
FULL = 2 * B
R3 = 512
NB3 = B // R3
EPS = 1e-12


def _dist(g, na_col, nb_row):
    return jnp.sqrt(jnp.maximum(na_col + nb_row - 2.0 * g, 0.0) + EPS)


def _matmul_nt(a, b):
    return jax.lax.dot_general(
        a, b, (((1,), (1,)), ((), ())),
        preferred_element_type=jnp.float32,
        precision=jax.lax.Precision.DEFAULT)


_LO_BITS = 0x358637BD
_HI_BITS = 0x40100000
_N_BISECT = 28


def _rank_value2(bits, k_a, k_b):
    r = bits.shape[0]
    lo_a = jnp.full((r, 1), jnp.int32(_LO_BITS))
    hi_a = jnp.full((r, 1), jnp.int32(_HI_BITS))
    lo_b = lo_a
    hi_b = hi_a

    def body(_, carry):
        lo_a, hi_a, lo_b, hi_b = carry
        mid_a = lo_a + ((hi_a - lo_a) >> 1)
        mid_b = lo_b + ((hi_b - lo_b) >> 1)
        cnt_a = jnp.sum((bits <= mid_a).astype(jnp.float32), axis=1,
                        keepdims=True)
        cnt_b = jnp.sum((bits <= mid_b).astype(jnp.float32), axis=1,
                        keepdims=True)
        p_a = cnt_a >= (k_a + 1.0)
        p_b = cnt_b >= (k_b + 1.0)
        return (jnp.where(p_a, lo_a, mid_a + 1), jnp.where(p_a, mid_a, hi_a),
                jnp.where(p_b, lo_b, mid_b + 1), jnp.where(p_b, mid_b, hi_b))

    lo_a, _, lo_b, _ = jax.lax.fori_loop(0, _N_BISECT, body,
                                         (lo_a, hi_a, lo_b, hi_b))
    return (jax.lax.bitcast_convert_type(lo_a, jnp.float32),
            jax.lax.bitcast_convert_type(lo_b, jnp.float32))


def _lid2(d, self_mask):
    bits = jax.lax.bitcast_convert_type(d, jnp.int32)
    t32v, t512v = _rank_value2(bits, 32.0, 512.0)
    logd = jnp.log(d)
    s32 = jnp.sum(jnp.where(d <= t32v, logd, 0.0), axis=1)
    c32 = jnp.sum((d <= t32v).astype(jnp.float32), axis=1)
    s512 = jnp.sum(jnp.where(d <= t512v, logd, 0.0), axis=1)
    c512 = jnp.sum((d <= t512v).astype(jnp.float32), axis=1)
    dself = jnp.sum(jnp.where(self_mask, d, 0.0), axis=1)
    lself = jnp.log(dself)
    tot32 = (s32 - lself) - (c32 - 1.0) * jnp.log(t32v[:, 0])
    tot512 = (s512 - lself) - (c512 - 1.0) * jnp.log(t512v[:, 0])
    return -32.0 / tot32, -512.0 / tot512


def _extract(dmat, n_extract, want_idx):
    r, n = dmat.shape
    iota = jax.lax.broadcasted_iota(jnp.int32, (r, n), 1)
    vals, idxs = [], []
    cur = dmat
    for _ in range(n_extract):
        mv = jnp.min(cur, axis=1, keepdims=True)
        mask = cur == mv
        vals.append(mv)
        if want_idx:
            cand = jnp.where(mask, iota, n)
            idxs.append(jnp.min(cand, axis=1, keepdims=True))
        cur = jnp.where(mask, jnp.inf, cur)
    return vals, idxs


def _img_kernel(ls_ref, x_ref, t_ref, xb_ref,
                logits_ref, lse_row_ref, diag_ref, s16_ref, s17_ref,
                nbr_ref, v32_ref, v512_ref, lse_col_ref,
                colm_ref, cols_ref):
    i = pl.program_id(0)
    X = x_ref[:]
    T = t_ref[:]
    xb = xb_ref[:]
    ls = ls_ref[0]
    nx = jnp.sum(X * X, axis=1)[None, :]
    nt = jnp.sum(T * T, axis=1)[None, :]
    nxb = jnp.sum(xb * xb, axis=1)[:, None]

    g_ii = _matmul_nt(xb, X)
    g_it = _matmul_nt(xb, T)
    d_ii = _dist(g_ii, nxb, nx)
    d_it = _dist(g_it, nxb, nt)

    logits = ls * g_it
    logits_ref[:] = logits

    m = jnp.max(logits, axis=1, keepdims=True)
    lse_row_ref[:] = m[:, 0] + jnp.log(jnp.sum(jnp.exp(logits - m), axis=1))

    @pl.when(i == 0)
    def _():
        colm_ref[:] = jnp.full((1, B), -jnp.inf, jnp.float32)
        cols_ref[:] = jnp.zeros((1, B), jnp.float32)

    bm = jnp.max(logits, axis=0, keepdims=True)
    newm = jnp.maximum(colm_ref[:], bm)
    news = (cols_ref[:] * jnp.exp(colm_ref[:] - newm)
            + jnp.sum(jnp.exp(logits - newm), axis=0, keepdims=True))
    colm_ref[:] = newm
    cols_ref[:] = news

    @pl.when(i == NBLK - 1)
    def _():
        lse_col_ref[:] = (newm + jnp.log(news))[0]

    rows = jax.lax.broadcasted_iota(jnp.int32, (RB, B), 0) + i * RB
    colsb = jax.lax.broadcasted_iota(jnp.int32, (RB, B), 1)
    self_mask = colsb == rows
    diag_ref[:] = jnp.sum(jnp.where(self_mask, logits, 0.0), axis=1)

    dfull = jnp.concatenate([jnp.where(self_mask, jnp.inf, d_ii), d_it],
                            axis=1)
    vals, _ = _extract(dfull, 17, False)
    s16v = vals[15]
    s16_ref[:] = s16v[:, 0]
    s17_ref[:] = vals[16][:, 0]
    nbr_ref[:] = (dfull <= s16v).astype(jnp.int8)

    v32, v512 = _lid2(d_ii, self_mask)
    v32_ref[:] = v32
    v512_ref[:] = v512


def _txt_kernel(x_ref, t_ref, tb_ref, s16_ref, t32_ref, t512_ref):
    i = pl.program_id(0)
    X = x_ref[:]
    T = t_ref[:]
    tb = tb_ref[:]
    nx = jnp.sum(X * X, axis=1)[None, :]
    nt = jnp.sum(T * T, axis=1)[None, :]
    ntb = jnp.sum(tb * tb, axis=1)[:, None]

    g_ti = _matmul_nt(tb, X)
    g_tt = _matmul_nt(tb, T)
    d_ti = _dist(g_ti, ntb, nx)
    d_tt = _dist(g_tt, ntb, nt)

    rows = jax.lax.broadcasted_iota(jnp.int32, (RB, B), 0) + i * RB
    colsb = jax.lax.broadcasted_iota(jnp.int32, (RB, B), 1)
    self_mask = colsb == rows

    dfull = jnp.concatenate([d_ti, jnp.where(self_mask, jnp.inf, d_tt)],
                            axis=1)
    vals, _ = _extract(dfull, 16, False)
    s16_ref[:] = vals[15][:, 0]

    t32, t512 = _lid2(d_tt, self_mask)
    t32_ref[:] = t32
    t512_ref[:] = t512


def _final_kernel(tf_ref, tfs_ref, nbr_ref, s17_ref, lser_ref, lsec_ref,
                  diag_ref, scores_ref, loss_ref, acc_ref):
    i = pl.program_id(0)

    @pl.when(i == 0)
    def _():
        acc_ref[0] = 0.0
        acc_ref[1] = 0.0
        acc_ref[2] = 0.0

    w = (1.0 / tf_ref[:])[None, :]
    ws = (1.0 / tfs_ref[:])[None, :]
    mask = nbr_ref[:].astype(jnp.float32)
    rows = jax.lax.broadcasted_iota(jnp.int32, (R3, FULL), 0) + i * R3
    cols = jax.lax.broadcasted_iota(jnp.int32, (R3, FULL), 1)
    wsel = jnp.where(cols > rows, ws, w)
    asum = jnp.sum(mask * wsel, axis=1)
    scores = s17_ref[:] * asum * (1.0 / K)
    scores_ref[:] = scores

    acc_ref[0] += jnp.sum(lser_ref[:] - diag_ref[:])
    acc_ref[1] += jnp.sum(lsec_ref[:] - diag_ref[:])
    acc_ref[2] += jnp.sum(scores)

    @pl.when(i == NB3 - 1)
    def _():
        sim = (acc_ref[0] + acc_ref[1]) / (2.0 * B)
        loss_ref[:, :] = jnp.full((1, 1), sim + acc_ref[2] / B, jnp.float32)


def _row_spec(blk):
    return pl.BlockSpec((blk,), lambda i: (i,))


def kernel(image_features, text_features, logit_scale):
    X = image_features
    T = text_features
    ls = logit_scale.astype(jnp.float32)

    full_spec = pl.BlockSpec((B, D), lambda i: (0, 0))
    blk_spec = pl.BlockSpec((RB, D), lambda i: (i, 0))

    out1 = pl.pallas_call(
        _img_kernel,
        grid=(NBLK,),
        in_specs=[
            pl.BlockSpec(memory_space=pltpu.SMEM),
            full_spec, full_spec, blk_spec,
        ],
        out_specs=[
            pl.BlockSpec((RB, B), lambda i: (i, 0)),
            _row_spec(RB), _row_spec(RB),
            _row_spec(RB), _row_spec(RB),
            pl.BlockSpec((RB, FULL), lambda i: (i, 0)),
            _row_spec(RB), _row_spec(RB),
            pl.BlockSpec((B,), lambda i: (0,)),
        ],
        out_shape=[
            jax.ShapeDtypeStruct((B, B), jnp.float32),
            jax.ShapeDtypeStruct((B,), jnp.float32),
            jax.ShapeDtypeStruct((B,), jnp.float32),
            jax.ShapeDtypeStruct((B,), jnp.float32),
            jax.ShapeDtypeStruct((B,), jnp.float32),
            jax.ShapeDtypeStruct((B, FULL), jnp.int8),
            jax.ShapeDtypeStruct((B,), jnp.float32),
            jax.ShapeDtypeStruct((B,), jnp.float32),
            jax.ShapeDtypeStruct((B,), jnp.float32),
        ],
        scratch_shapes=[
            pltpu.VMEM((1, B), jnp.float32),
            pltpu.VMEM((1, B), jnp.float32),
        ],
        compiler_params=pltpu.CompilerParams(
            dimension_semantics=("arbitrary",)),
    )(ls, X, T, X)
    (logits, lse_row, diag, s16_img, s17, nbr, v32, v512, lse_col) = out1

    out2 = pl.pallas_call(
        _txt_kernel,
        grid=(NBLK,),
        in_specs=[full_spec, full_spec, blk_spec],
        out_specs=[_row_spec(RB), _row_spec(RB), _row_spec(RB)],
        out_shape=[
            jax.ShapeDtypeStruct((B,), jnp.float32),
            jax.ShapeDtypeStruct((B,), jnp.float32),
            jax.ShapeDtypeStruct((B,), jnp.float32),
        ],
        compiler_params=pltpu.CompilerParams(
            dimension_semantics=("arbitrary",)),
    )(X, T, T)
    s16_txt, t32, t512 = out2

    t_full = jnp.concatenate([s16_img, s16_txt])
    t_full_s = jnp.concatenate([t_full[:1], t_full[:-1]])

    scores, loss2d = pl.pallas_call(
        _final_kernel,
        grid=(NB3,),
        in_specs=[
            pl.BlockSpec((FULL,), lambda i: (0,)),
            pl.BlockSpec((FULL,), lambda i: (0,)),
            pl.BlockSpec((R3, FULL), lambda i: (i, 0)),
            _row_spec(R3), _row_spec(R3), _row_spec(R3), _row_spec(R3),
        ],
        out_specs=[
            _row_spec(R3),
            pl.BlockSpec((1, 1), lambda i: (0, 0)),
        ],
        out_shape=[
            jax.ShapeDtypeStruct((B,), jnp.float32),
            jax.ShapeDtypeStruct((1, 1), jnp.float32),
        ],
        scratch_shapes=[pltpu.SMEM((4,), jnp.float32)],
        compiler_params=pltpu.CompilerParams(
            dimension_semantics=("arbitrary",)),
    )(t_full, t_full_s, nbr, s17, lse_row, lse_col, diag)

    labels = jnp.arange(B, dtype=jnp.int32)
    return (loss2d.reshape(()), logits, labels, scores, v32, v512, t32, t512)
